# baseline (speedup 1.0000x reference)
import functools

import jax
import jax.numpy as jnp
import numpy as np
from jax import lax
from jax.experimental import pallas as pl
from jax.experimental.pallas import tpu as pltpu
from jax.experimental.pallas import tpu_sc as plsc

N = 10000
E = 320000
D_IN = 128
H = 8
DH = 16
D_OUT = H * DH
DW = D_OUT + 2 * H
NEG_SLOPE = 0.2

C = 80
NTILE = 16
NCHUNK = E // C
CH_PER_TILE = NCHUNK // NTILE
ROWS_PT = 624
ROWS_REM = N - ROWS_PT * NTILE


def _tc_front_body(x_ref, ws_ref, wsp_ref, crs_ref, crd_ref, wp_ref, bp_ref,
                   rel_ref, feat_ref, fbf_ref, esd_ref, edd_ref, relo_ref):
    x = x_ref[...]
    feat = jnp.dot(x, ws_ref[0], preferred_element_type=jnp.float32)
    feat_ref[...] = feat
    fbf_ref[...] = jnp.dot(x, wsp_ref[0],
                           preferred_element_type=jnp.float32
                           ).astype(jnp.bfloat16)
    cs = jnp.dot(rel_ref[0], crs_ref[0], preferred_element_type=jnp.float32)
    cd = jnp.dot(rel_ref[0], crd_ref[0], preferred_element_type=jnp.float32)
    i = lax.broadcasted_iota(jnp.int32, (D_OUT, 2 * H), 0)
    k = lax.broadcasted_iota(jnp.int32, (D_OUT, 2 * H), 1)
    m = ((i // DH) == (k % H)).astype(jnp.float32)
    esd_ref[...] = jnp.dot(feat * cs, m, preferred_element_type=jnp.float32)
    edd_ref[...] = jnp.dot(feat * cd, m, preferred_element_type=jnp.float32)
    relo_ref[...] = (jnp.dot(rel_ref[0], wp_ref[0],
                             preferred_element_type=jnp.float32)
                     + bp_ref[0])[None]


def _tc_front(x, ws, wsp, crs, crd, wp, bp, rel):
    return pl.pallas_call(
        _tc_front_body,
        grid=(2,),
        in_specs=[
            pl.BlockSpec((N, D_IN), lambda g: (0, 0)),
            pl.BlockSpec((1, D_IN, D_OUT), lambda g: (g, 0, 0)),
            pl.BlockSpec((1, D_IN, D_OUT), lambda g: (g, 0, 0)),
            pl.BlockSpec((1, 16, D_OUT), lambda g: (g, 0, 0)),
            pl.BlockSpec((1, 16, D_OUT), lambda g: (g, 0, 0)),
            pl.BlockSpec((1, 16, D_OUT), lambda g: (g, 0, 0)),
            pl.BlockSpec((1, 1, D_OUT), lambda g: (g, 0, 0)),
            pl.BlockSpec((1, 1, 16), lambda g: (g, 0, 0)),
        ],
        out_specs=[
            pl.BlockSpec((N, D_OUT), lambda g: (g, 0)),
            pl.BlockSpec((N, D_OUT), lambda g: (g, 0)),
            pl.BlockSpec((N, 2 * H), lambda g: (g, 0)),
            pl.BlockSpec((N, 2 * H), lambda g: (g, 0)),
            pl.BlockSpec((1, 1, D_OUT), lambda g: (g, 0, 0)),
        ],
        out_shape=[
            jax.ShapeDtypeStruct((2 * N, D_OUT), jnp.float32),
            jax.ShapeDtypeStruct((2 * N, D_OUT), jnp.bfloat16),
            jax.ShapeDtypeStruct((2 * N, 2 * H), jnp.float32),
            jax.ShapeDtypeStruct((2 * N, 2 * H), jnp.float32),
            jax.ShapeDtypeStruct((2, 1, D_OUT), jnp.float32),
        ],
    )(x, ws, wsp, crs, crd, wp, bp, rel)


_GDN = lax.GatherDimensionNumbers(
    offset_dims=(), collapsed_slice_dims=(0,), start_index_map=(0,))


def _bcast16(v, h):
    idx = jnp.full((16, 1), h, dtype=jnp.int32)
    return lax.gather(v, idx, _GDN, (1,),
                      mode=lax.GatherScatterMode.PROMISE_IN_BOUNDS)


def _sc_edge_body(g_hbm, edd_hbm, src_hbm, dst_hbm, zero_hbm,
                  acc_out,
                  sraw0, draw0, sraw1, draw1, srcg, dstg, dstl,
                  gb0, edb0, mb0, gb1, edb1, mb1,
                  sem_i, sem_g, sem_s, acc_sp):
    c = lax.axis_index("c")
    s = lax.axis_index("s")
    r0 = s * ROWS_PT
    pltpu.sync_copy(zero_hbm.at[pl.ds(r0, ROWS_PT)],
                    acc_sp.at[pl.ds(r0, ROWS_PT)])

    @pl.when(s == NTILE - 1)
    def _zero_rem():
        rr = ROWS_PT * NTILE
        pltpu.sync_copy(zero_hbm.at[pl.ds(rr, ROWS_REM)],
                        acc_sp.at[pl.ds(rr, ROWS_REM)])

    plsc.subcore_barrier()

    coff = c * N
    row0 = c * NCHUNK + s * CH_PER_TILE

    def fire_idx(j, sr, dr):
        pltpu.async_copy(src_hbm.at[row0 + j], sr, sem_i)
        pltpu.async_copy(dst_hbm.at[row0 + j], dr, sem_i)

    def drain_idx(sr, dr):
        pltpu.make_async_copy(src_hbm.at[0], sr, sem_i).wait()
        pltpu.make_async_copy(dst_hbm.at[0], dr, sem_i).wait()

    def fire_gather(gb, edb):
        pltpu.async_copy(g_hbm.at[srcg], gb, sem_g)
        pltpu.async_copy(edd_hbm.at[dstg], edb, sem_g)

    def drain_gather(gb, edb):
        pltpu.make_async_copy(g_hbm.at[srcg], gb, sem_g).wait()
        pltpu.make_async_copy(edd_hbm.at[dstg], edb, sem_g).wait()

    def globalize(sr, dr):
        for k in range(C // 16):
            sl = pl.ds(k * 16, 16)
            srcg[sl] = sr[sl] + coff
            dstg[sl] = dr[sl] + coff

    def localize(dr):
        for k in range(C // 16):
            sl = pl.ds(k * 16, 16)
            dstl[sl] = dr[sl]

    def process(j, raw_b, gb_b, mb_b, raw_n, gb_n, mb_n):
        sr_b, dr_b = raw_b
        gbuf_b, edb_b = gb_b
        sr_n, dr_n = raw_n
        gbuf_n, edb_n = gb_n

        drain_gather(gbuf_b, edb_b)

        @pl.when(j >= 1)
        def _():
            pltpu.make_async_copy(mb_n, acc_sp.at[dstl], sem_s).wait()

        localize(dr_b)

        @pl.when(j + 1 < CH_PER_TILE)
        def _():
            drain_idx(sr_n, dr_n)
            globalize(sr_n, dr_n)
            fire_gather(gbuf_n, edb_n)

        @pl.when(j + 2 < CH_PER_TILE)
        def _():
            fire_idx(j + 2, sr_b, dr_b)

        @plsc.parallel_loop(0, C, 1, unroll=2)
        def row(r):
            es = plsc.bitcast(gbuf_b[r, pl.ds(D_OUT, 32)], jnp.float32)
            e = es + edb_b[r, :]
            e = jnp.where(e >= 0, e, NEG_SLOPE * e)
            ee = jnp.exp(e)
            mb_b[r, pl.ds(D_OUT, 16)] = ee
            for h2 in range(H // 2):
                v = gbuf_b[r, pl.ds(h2 * 32, 32)]
                a, b = plsc.unpack(v, format=plsc.PackFormat.INTERLEAVED)
                sla = pl.ds((2 * h2) * DH, DH)
                slb = pl.ds((2 * h2 + 1) * DH, DH)
                mb_b[r, sla] = a * _bcast16(ee, 2 * h2)
                mb_b[r, slb] = b * _bcast16(ee, 2 * h2 + 1)

        pltpu.async_copy(mb_b, acc_sp.at[dstl], sem_s, add=True)

    fire_idx(0, sraw0, draw0)
    drain_idx(sraw0, draw0)
    globalize(sraw0, draw0)
    fire_gather(gb0, edb0)
    fire_idx(1, sraw1, draw1)

    def pair(i, carry):
        j = 2 * i
        process(j, (sraw0, draw0), (gb0, edb0), mb0,
                (sraw1, draw1), (gb1, edb1), mb1)
        process(j + 1, (sraw1, draw1), (gb1, edb1), mb1,
                (sraw0, draw0), (gb0, edb0), mb0)
        return carry

    lax.fori_loop(0, CH_PER_TILE // 2, pair, 0)
    pltpu.make_async_copy(mb1, acc_sp.at[dstl], sem_s).wait()

    plsc.subcore_barrier()
    pltpu.sync_copy(acc_sp.at[pl.ds(r0, ROWS_PT)],
                    acc_out.at[pl.ds(coff + r0, ROWS_PT)])

    @pl.when(s == NTILE - 1)
    def _out_rem():
        rr = ROWS_PT * NTILE
        pltpu.sync_copy(acc_sp.at[pl.ds(rr, ROWS_REM)],
                        acc_out.at[pl.ds(coff + rr, ROWS_REM)])


@functools.cache
def _sc_edge():
  return pl.kernel(
    _sc_edge_body,
    out_type=[
        jax.ShapeDtypeStruct((2 * N, DW), jnp.float32),
    ],
    mesh=plsc.VectorSubcoreMesh(core_axis_name="c", subcore_axis_name="s"),
    compiler_params=pltpu.CompilerParams(use_tc_tiling_on_sc=False,
                                         needs_layout_passes=False),
    scratch_types=[
        pltpu.VMEM((C,), jnp.int32),
        pltpu.VMEM((C,), jnp.int32),
        pltpu.VMEM((C,), jnp.int32),
        pltpu.VMEM((C,), jnp.int32),
        pltpu.VMEM((C,), jnp.int32),
        pltpu.VMEM((C,), jnp.int32),
        pltpu.VMEM((C,), jnp.int32),
        pltpu.VMEM((C, D_OUT + 32), jnp.bfloat16),
        pltpu.VMEM((C, 2 * H), jnp.float32),
        pltpu.VMEM((C, DW), jnp.float32),
        pltpu.VMEM((C, D_OUT + 32), jnp.bfloat16),
        pltpu.VMEM((C, 2 * H), jnp.float32),
        pltpu.VMEM((C, DW), jnp.float32),
        pltpu.SemaphoreType.DMA,
        pltpu.SemaphoreType.DMA,
        pltpu.SemaphoreType.DMA,
        pltpu.VMEM_SHARED((N, DW), jnp.float32),
    ],
  )


def _tc_epi_body(acc_ref, out_ref):
    a = acc_ref[...]
    for h in range(H):
        zz = a[:, D_OUT + h:D_OUT + h + 1]
        inv = jnp.where(zz == 0.0, 0.0, 1.0 / zz)
        out_ref[:, h * DH:(h + 1) * DH] = jnp.maximum(
            a[:, h * DH:(h + 1) * DH] * inv, 0.0)


def _tc_epi(acc):
    return pl.pallas_call(
        _tc_epi_body,
        grid=(2,),
        in_specs=[pl.BlockSpec((N, DW), lambda g: (g, 0))],
        out_specs=pl.BlockSpec((N, D_OUT), lambda g: (g, 0)),
        out_shape=jax.ShapeDtypeStruct((2 * N, D_OUT), jnp.float32),
    )(acc)


_PERM_SRC = np.asarray([h * 2 * DH + d for h in range(H) for d in range(DH)])
_PERM_DST = _PERM_SRC + DH
_PERM_BF = np.asarray([(2 * (q // 32) + (q % 2)) * DH + (q % 32) // 2
                       for q in range(D_OUT)])


def kernel(x, edge_index_e0, edge_index_e1, rel_emb_e0, rel_emb_e1,
           W_src_e0, W_src_e1, W_rel_e0, W_rel_e1,
           W_prop_e0, W_prop_e1, b_prop_e0, b_prop_e1):
    ws = jnp.stack([W_src_e0, W_src_e1])
    wsp = jnp.stack([W_src_e0[:, _PERM_BF], W_src_e1[:, _PERM_BF]])
    crs = jnp.stack([W_rel_e0[:, _PERM_SRC], W_rel_e1[:, _PERM_SRC]])
    crd = jnp.stack([W_rel_e0[:, _PERM_DST], W_rel_e1[:, _PERM_DST]])
    wp = jnp.stack([W_prop_e0, W_prop_e1])
    bp = jnp.stack([b_prop_e0, b_prop_e1])[:, None, :]
    rel = jnp.stack([rel_emb_e0, rel_emb_e1])[:, None, :]

    feat, fbf, esd, edd, relo = _tc_front(x, ws, wsp, crs, crd, wp, bp, rel)

    src = jnp.concatenate([edge_index_e0[0], edge_index_e1[0]]).reshape(
        2 * NCHUNK, C)
    dst = jnp.concatenate([edge_index_e0[1], edge_index_e1[1]]).reshape(
        2 * NCHUNK, C)
    zero = jnp.zeros((N, DW), jnp.float32)
    es_bits = lax.bitcast_convert_type(esd, jnp.bfloat16).reshape(2 * N, 32)
    g = jnp.concatenate([fbf, es_bits], axis=1)

    (acc,) = _sc_edge()(g, edd, src, dst, zero)
    out = _tc_epi(acc)

    rel0 = relo[0, 0]
    rel1 = relo[1, 0]
    feat0 = feat[:N].reshape(N, H, DH)
    feat1 = feat[N:].reshape(N, H, DH)
    return out[:N], out[N:], rel0, rel1, feat0, feat1


# speedup vs baseline: 1.1300x; 1.1300x over previous
import functools

import jax
import jax.numpy as jnp
import numpy as np
from jax import lax
from jax.experimental import pallas as pl
from jax.experimental.pallas import tpu as pltpu
from jax.experimental.pallas import tpu_sc as plsc

N = 10000
E = 320000
D_IN = 128
H = 8
DH = 16
D_OUT = H * DH
DW = D_OUT + 2 * H
NEG_SLOPE = 0.2

C = 80
NTILE = 16
NCHUNK = E // C
CH_PER_TILE = NCHUNK // NTILE
ROWS_PT = 624
ROWS_REM = N - ROWS_PT * NTILE


def _tc_front_body(x_ref, ws_ref, wsp_ref, crs_ref, crd_ref, wp_ref, bp_ref,
                   rel_ref, feat_ref, fbf_ref, esd_ref, edd_ref, relo_ref):
    x = x_ref[...]
    feat = jnp.dot(x, ws_ref[0], preferred_element_type=jnp.float32)
    feat_ref[...] = feat
    fbf_ref[...] = jnp.dot(x, wsp_ref[0],
                           preferred_element_type=jnp.float32
                           ).astype(jnp.bfloat16)
    cs = jnp.dot(rel_ref[0], crs_ref[0], preferred_element_type=jnp.float32)
    cd = jnp.dot(rel_ref[0], crd_ref[0], preferred_element_type=jnp.float32)
    i = lax.broadcasted_iota(jnp.int32, (D_OUT, 2 * H), 0)
    k = lax.broadcasted_iota(jnp.int32, (D_OUT, 2 * H), 1)
    m = ((i // DH) == (k % H)).astype(jnp.float32)
    esd_ref[...] = jnp.dot(feat * cs, m, preferred_element_type=jnp.float32)
    edd_ref[...] = jnp.dot(feat * cd, m, preferred_element_type=jnp.float32)
    relo_ref[...] = (jnp.dot(rel_ref[0], wp_ref[0],
                             preferred_element_type=jnp.float32)
                     + bp_ref[0])[None]


def _tc_front(x, ws, wsp, crs, crd, wp, bp, rel):
    return pl.pallas_call(
        _tc_front_body,
        grid=(2,),
        in_specs=[
            pl.BlockSpec((N, D_IN), lambda g: (0, 0)),
            pl.BlockSpec((1, D_IN, D_OUT), lambda g: (g, 0, 0)),
            pl.BlockSpec((1, D_IN, D_OUT), lambda g: (g, 0, 0)),
            pl.BlockSpec((1, 16, D_OUT), lambda g: (g, 0, 0)),
            pl.BlockSpec((1, 16, D_OUT), lambda g: (g, 0, 0)),
            pl.BlockSpec((1, 16, D_OUT), lambda g: (g, 0, 0)),
            pl.BlockSpec((1, 1, D_OUT), lambda g: (g, 0, 0)),
            pl.BlockSpec((1, 1, 16), lambda g: (g, 0, 0)),
        ],
        out_specs=[
            pl.BlockSpec((N, D_OUT), lambda g: (g, 0)),
            pl.BlockSpec((N, D_OUT), lambda g: (g, 0)),
            pl.BlockSpec((N, 2 * H), lambda g: (g, 0)),
            pl.BlockSpec((N, 2 * H), lambda g: (g, 0)),
            pl.BlockSpec((1, 1, D_OUT), lambda g: (g, 0, 0)),
        ],
        out_shape=[
            jax.ShapeDtypeStruct((2 * N, D_OUT), jnp.float32),
            jax.ShapeDtypeStruct((2 * N, D_OUT), jnp.bfloat16),
            jax.ShapeDtypeStruct((2 * N, 2 * H), jnp.float32),
            jax.ShapeDtypeStruct((2 * N, 2 * H), jnp.float32),
            jax.ShapeDtypeStruct((2, 1, D_OUT), jnp.float32),
        ],
    )(x, ws, wsp, crs, crd, wp, bp, rel)


_GDN = lax.GatherDimensionNumbers(
    offset_dims=(), collapsed_slice_dims=(0,), start_index_map=(0,))


def _bcast16(v, h):
    idx = jnp.full((16, 1), h, dtype=jnp.int32)
    return lax.gather(v, idx, _GDN, (1,),
                      mode=lax.GatherScatterMode.PROMISE_IN_BOUNDS)


def _sc_edge_body(fbf_hbm, esd_hbm, edd_hbm, src_hbm, dst_hbm, zero_hbm,
                  acc_out,
                  sraw0, draw0, sraw1, draw1, srcg, dstg, dstl,
                  ff0, esb0, edb0, mb0, ff1, esb1, edb1, mb1,
                  sem_i, sem_g, sem_s, acc_sp):
    c = lax.axis_index("c")
    s = lax.axis_index("s")
    r0 = s * ROWS_PT
    pltpu.sync_copy(zero_hbm.at[pl.ds(r0, ROWS_PT)],
                    acc_sp.at[pl.ds(r0, ROWS_PT)])

    @pl.when(s == NTILE - 1)
    def _zero_rem():
        rr = ROWS_PT * NTILE
        pltpu.sync_copy(zero_hbm.at[pl.ds(rr, ROWS_REM)],
                        acc_sp.at[pl.ds(rr, ROWS_REM)])

    plsc.subcore_barrier()

    coff = c * N
    row0 = c * NCHUNK + s * CH_PER_TILE

    def fire_idx(j, sr, dr):
        pltpu.async_copy(src_hbm.at[row0 + j], sr, sem_i)
        pltpu.async_copy(dst_hbm.at[row0 + j], dr, sem_i)

    def drain_idx(sr, dr):
        pltpu.make_async_copy(src_hbm.at[0], sr, sem_i).wait()
        pltpu.make_async_copy(dst_hbm.at[0], dr, sem_i).wait()

    def fire_gather(ff, esb, edb):
        pltpu.async_copy(fbf_hbm.at[srcg], ff, sem_g)
        pltpu.async_copy(esd_hbm.at[srcg], esb, sem_g)
        pltpu.async_copy(edd_hbm.at[dstg], edb, sem_g)

    def drain_gather(ff, esb, edb):
        pltpu.make_async_copy(fbf_hbm.at[srcg], ff, sem_g).wait()
        pltpu.make_async_copy(esd_hbm.at[srcg], esb, sem_g).wait()
        pltpu.make_async_copy(edd_hbm.at[dstg], edb, sem_g).wait()

    def globalize(sr, dr):
        for k in range(C // 16):
            sl = pl.ds(k * 16, 16)
            srcg[sl] = sr[sl] + coff
            dstg[sl] = dr[sl] + coff

    def localize(dr):
        for k in range(C // 16):
            sl = pl.ds(k * 16, 16)
            dstl[sl] = dr[sl]

    def process(j, raw_b, gb_b, mb_b, raw_n, gb_n, mb_n):
        sr_b, dr_b = raw_b
        ff_b, esb_b, edb_b = gb_b
        sr_n, dr_n = raw_n
        ff_n, esb_n, edb_n = gb_n

        drain_gather(ff_b, esb_b, edb_b)

        @pl.when(j >= 1)
        def _():
            pltpu.make_async_copy(mb_n, acc_sp.at[dstl], sem_s).wait()

        localize(dr_b)

        @pl.when(j + 1 < CH_PER_TILE)
        def _():
            drain_idx(sr_n, dr_n)
            globalize(sr_n, dr_n)
            fire_gather(ff_n, esb_n, edb_n)

        @pl.when(j + 2 < CH_PER_TILE)
        def _():
            fire_idx(j + 2, sr_b, dr_b)

        @plsc.parallel_loop(0, C, 1, unroll=2)
        def row(r):
            e = esb_b[r, :] + edb_b[r, :]
            e = jnp.where(e >= 0, e, NEG_SLOPE * e)
            ee = jnp.exp(e)
            mb_b[r, pl.ds(D_OUT, 16)] = ee
            for h2 in range(H // 2):
                v = ff_b[r, pl.ds(h2 * 32, 32)]
                a, b = plsc.unpack(v, format=plsc.PackFormat.INTERLEAVED)
                sla = pl.ds((2 * h2) * DH, DH)
                slb = pl.ds((2 * h2 + 1) * DH, DH)
                mb_b[r, sla] = a * _bcast16(ee, 2 * h2)
                mb_b[r, slb] = b * _bcast16(ee, 2 * h2 + 1)

        pltpu.async_copy(mb_b, acc_sp.at[dstl], sem_s, add=True)

    fire_idx(0, sraw0, draw0)
    drain_idx(sraw0, draw0)
    globalize(sraw0, draw0)
    fire_gather(ff0, esb0, edb0)
    fire_idx(1, sraw1, draw1)

    def pair(i, carry):
        j = 2 * i
        process(j, (sraw0, draw0), (ff0, esb0, edb0), mb0,
                (sraw1, draw1), (ff1, esb1, edb1), mb1)
        process(j + 1, (sraw1, draw1), (ff1, esb1, edb1), mb1,
                (sraw0, draw0), (ff0, esb0, edb0), mb0)
        return carry

    lax.fori_loop(0, CH_PER_TILE // 2, pair, 0)
    pltpu.make_async_copy(mb1, acc_sp.at[dstl], sem_s).wait()

    plsc.subcore_barrier()
    pltpu.sync_copy(acc_sp.at[pl.ds(r0, ROWS_PT)],
                    acc_out.at[pl.ds(coff + r0, ROWS_PT)])

    @pl.when(s == NTILE - 1)
    def _out_rem():
        rr = ROWS_PT * NTILE
        pltpu.sync_copy(acc_sp.at[pl.ds(rr, ROWS_REM)],
                        acc_out.at[pl.ds(coff + rr, ROWS_REM)])


@functools.cache
def _sc_edge():
  return pl.kernel(
    _sc_edge_body,
    out_type=[
        jax.ShapeDtypeStruct((2 * N, DW), jnp.float32),
    ],
    mesh=plsc.VectorSubcoreMesh(core_axis_name="c", subcore_axis_name="s"),
    compiler_params=pltpu.CompilerParams(use_tc_tiling_on_sc=False,
                                         needs_layout_passes=False),
    scratch_types=[
        pltpu.VMEM((C,), jnp.int32),
        pltpu.VMEM((C,), jnp.int32),
        pltpu.VMEM((C,), jnp.int32),
        pltpu.VMEM((C,), jnp.int32),
        pltpu.VMEM((C,), jnp.int32),
        pltpu.VMEM((C,), jnp.int32),
        pltpu.VMEM((C,), jnp.int32),
        pltpu.VMEM((C, D_OUT), jnp.bfloat16),
        pltpu.VMEM((C, 2 * H), jnp.float32),
        pltpu.VMEM((C, 2 * H), jnp.float32),
        pltpu.VMEM((C, DW), jnp.float32),
        pltpu.VMEM((C, D_OUT), jnp.bfloat16),
        pltpu.VMEM((C, 2 * H), jnp.float32),
        pltpu.VMEM((C, 2 * H), jnp.float32),
        pltpu.VMEM((C, DW), jnp.float32),
        pltpu.SemaphoreType.DMA,
        pltpu.SemaphoreType.DMA,
        pltpu.SemaphoreType.DMA,
        pltpu.VMEM_SHARED((N, DW), jnp.float32),
    ],
  )


def _tc_epi_body(acc_ref, out_ref):
    a = acc_ref[...]
    for h in range(H):
        zz = a[:, D_OUT + h:D_OUT + h + 1]
        inv = jnp.where(zz == 0.0, 0.0, 1.0 / zz)
        out_ref[:, h * DH:(h + 1) * DH] = jnp.maximum(
            a[:, h * DH:(h + 1) * DH] * inv, 0.0)


def _tc_epi(acc):
    return pl.pallas_call(
        _tc_epi_body,
        grid=(2,),
        in_specs=[pl.BlockSpec((N, DW), lambda g: (g, 0))],
        out_specs=pl.BlockSpec((N, D_OUT), lambda g: (g, 0)),
        out_shape=jax.ShapeDtypeStruct((2 * N, D_OUT), jnp.float32),
    )(acc)


_PERM_SRC = np.asarray([h * 2 * DH + d for h in range(H) for d in range(DH)])
_PERM_DST = _PERM_SRC + DH
_PERM_BF = np.asarray([(2 * (q // 32) + (q % 2)) * DH + (q % 32) // 2
                       for q in range(D_OUT)])


def kernel(x, edge_index_e0, edge_index_e1, rel_emb_e0, rel_emb_e1,
           W_src_e0, W_src_e1, W_rel_e0, W_rel_e1,
           W_prop_e0, W_prop_e1, b_prop_e0, b_prop_e1):
    ws = jnp.stack([W_src_e0, W_src_e1])
    wsp = jnp.stack([W_src_e0[:, _PERM_BF], W_src_e1[:, _PERM_BF]])
    crs = jnp.stack([W_rel_e0[:, _PERM_SRC], W_rel_e1[:, _PERM_SRC]])
    crd = jnp.stack([W_rel_e0[:, _PERM_DST], W_rel_e1[:, _PERM_DST]])
    wp = jnp.stack([W_prop_e0, W_prop_e1])
    bp = jnp.stack([b_prop_e0, b_prop_e1])[:, None, :]
    rel = jnp.stack([rel_emb_e0, rel_emb_e1])[:, None, :]

    feat, fbf, esd, edd, relo = _tc_front(x, ws, wsp, crs, crd, wp, bp, rel)

    src = jnp.concatenate([edge_index_e0[0], edge_index_e1[0]]).reshape(
        2 * NCHUNK, C)
    dst = jnp.concatenate([edge_index_e0[1], edge_index_e1[1]]).reshape(
        2 * NCHUNK, C)
    zero = jnp.zeros((N, DW), jnp.float32)

    (acc,) = _sc_edge()(fbf, esd, edd, src, dst, zero)
    out = _tc_epi(acc)

    rel0 = relo[0, 0]
    rel1 = relo[1, 0]
    feat0 = feat[:N].reshape(N, H, DH)
    feat1 = feat[N:].reshape(N, H, DH)
    return out[:N], out[N:], rel0, rel1, feat0, feat1
